# baseline (speedup 1.0000x reference)
import jax
import jax.numpy as jnp
from jax.experimental import pallas as pl
from jax.experimental.pallas import tpu as pltpu

T, B, D_IN, D_H = 512, 64, 512, 512
G4 = 4 * D_H
BM = 1024
NC = 2
BC = B // NC


def _proj_kernel(x_ref, wxt_ref, b_ref, o_ref):
    o_ref[...] = (
        jnp.dot(x_ref[...], wxt_ref[...], preferred_element_type=jnp.float32)
        + b_ref[...]
    )


def _rec_kernel(pre_ref, wht_ref, out_ref, cx_ref, h_ref, c_ref):
    t = pl.program_id(1)

    @pl.when(t == 0)
    def _():
        h_ref[...] = jnp.zeros_like(h_ref)
        c_ref[...] = jnp.zeros_like(c_ref)

    gates = pre_ref[0] + jnp.dot(
        h_ref[...].astype(jnp.bfloat16),
        wht_ref[...],
        preferred_element_type=jnp.float32,
    )
    f = jax.nn.sigmoid(gates[:, 0 * D_H : 1 * D_H])
    i = jax.nn.sigmoid(gates[:, 1 * D_H : 2 * D_H])
    g = jnp.tanh(gates[:, 2 * D_H : 3 * D_H])
    o = jax.nn.sigmoid(gates[:, 3 * D_H : 4 * D_H])
    c_new = f * c_ref[...] + i * g
    h_new = o * jnp.tanh(c_new)
    c_ref[...] = c_new
    h_ref[...] = h_new
    out_ref[0] = h_new

    @pl.when(t == T - 1)
    def _():
        cx_ref[...] = c_new


def kernel(inputs, Wf, bf, Wi, bi, Wg, bg, Wo, bo):
    W = jnp.concatenate([Wf, Wi, Wg, Wo], axis=0)
    WxT = W[:, :D_IN].T
    WhT = W[:, D_IN:].T
    b = jnp.concatenate([bf, bi, bg, bo]).reshape(1, G4)

    X = inputs.reshape(T * B, D_IN).astype(jnp.bfloat16)
    WxT = WxT.astype(jnp.bfloat16)
    WhT = WhT.astype(jnp.bfloat16)
    pre = pl.pallas_call(
        _proj_kernel,
        out_shape=jax.ShapeDtypeStruct((T * B, G4), jnp.float32),
        grid=(T * B // BM,),
        in_specs=[
            pl.BlockSpec((BM, D_IN), lambda m: (m, 0)),
            pl.BlockSpec((D_IN, G4), lambda m: (0, 0)),
            pl.BlockSpec((1, G4), lambda m: (0, 0)),
        ],
        out_specs=pl.BlockSpec((BM, G4), lambda m: (m, 0)),
        compiler_params=pltpu.CompilerParams(
            dimension_semantics=("parallel",),
        ),
        name="lstm_proj",
    )(X, WxT, b)
    pre = pre.reshape(T, B, G4)

    outputs, cx = pl.pallas_call(
        _rec_kernel,
        out_shape=(
            jax.ShapeDtypeStruct((T, B, D_H), jnp.float32),
            jax.ShapeDtypeStruct((B, D_H), jnp.float32),
        ),
        grid=(NC, T),
        in_specs=[
            pl.BlockSpec((1, BC, G4), lambda c, t: (t, c, 0)),
            pl.BlockSpec((D_H, G4), lambda c, t: (0, 0)),
        ],
        out_specs=(
            pl.BlockSpec((1, BC, D_H), lambda c, t: (t, c, 0)),
            pl.BlockSpec((BC, D_H), lambda c, t: (c, 0)),
        ),
        scratch_shapes=[
            pltpu.VMEM((BC, D_H), jnp.float32),
            pltpu.VMEM((BC, D_H), jnp.float32),
        ],
        compiler_params=pltpu.CompilerParams(
            dimension_semantics=("parallel", "arbitrary"),
        ),
        name="lstm_rec",
    )(pre, WhT)

    hx = outputs[-1]
    return outputs, (hx, cx)


# speedup vs baseline: 2.4196x; 2.4196x over previous
import jax
import jax.numpy as jnp
from jax.experimental import pallas as pl
from jax.experimental.pallas import tpu as pltpu

T, B, D_IN, D_H = 512, 64, 512, 512
G4 = 4 * D_H
BM = 1024
MB = T * B // BM
S = 16


def _proj_kernel(x_ref, wxt_ref, b_ref, o_ref):
    o_ref[...] = (
        jnp.dot(x_ref[...], wxt_ref[...], preferred_element_type=jnp.float32)
        + b_ref[...]
    )


def _rec_kernel(pre_ref, wht_ref, out_ref, cx_ref, h_ref, c_ref):
    k = pl.program_id(0)

    @pl.when(k == 0)
    def _():
        h_ref[...] = jnp.zeros_like(h_ref)
        c_ref[...] = jnp.zeros_like(c_ref)

    h = h_ref[...]
    c = c_ref[...]
    wht = wht_ref[...]
    for s in range(S):
        gates = pre_ref[s] + jnp.dot(
            h, wht, preferred_element_type=jnp.float32
        )
        f = jax.nn.sigmoid(gates[:, 0 * D_H : 1 * D_H])
        i = jax.nn.sigmoid(gates[:, 1 * D_H : 2 * D_H])
        g = jnp.tanh(gates[:, 2 * D_H : 3 * D_H])
        o = jax.nn.sigmoid(gates[:, 3 * D_H : 4 * D_H])
        c = f * c + i * g
        h = o * jnp.tanh(c)
        out_ref[s] = h
    h_ref[...] = h
    c_ref[...] = c

    @pl.when(k == T // S - 1)
    def _():
        cx_ref[...] = c


def kernel(inputs, Wf, bf, Wi, bi, Wg, bg, Wo, bo):
    W = jnp.concatenate([Wf, Wi, Wg, Wo], axis=0)
    WxT = W[:, :D_IN].T
    WhT = W[:, D_IN:].T
    b = jnp.concatenate([bf, bi, bg, bo]).reshape(1, G4)

    X = inputs.reshape(T * B, D_IN)
    pre = pl.pallas_call(
        _proj_kernel,
        out_shape=jax.ShapeDtypeStruct((T * B, G4), jnp.float32),
        grid=(MB,),
        in_specs=[
            pl.BlockSpec((BM, D_IN), lambda m: (m, 0)),
            pl.BlockSpec((D_IN, G4), lambda m: (0, 0)),
            pl.BlockSpec((1, G4), lambda m: (0, 0)),
        ],
        out_specs=pl.BlockSpec((BM, G4), lambda m: (m, 0)),
        compiler_params=pltpu.CompilerParams(
            dimension_semantics=("parallel",),
        ),
        name="lstm_proj",
    )(X, WxT, b)
    pre = pre.reshape(T, B, G4)

    outputs, cx = pl.pallas_call(
        _rec_kernel,
        out_shape=(
            jax.ShapeDtypeStruct((T, B, D_H), jnp.float32),
            jax.ShapeDtypeStruct((B, D_H), jnp.float32),
        ),
        grid=(T // S,),
        in_specs=[
            pl.BlockSpec((S, B, G4), lambda k: (k, 0, 0)),
            pl.BlockSpec((D_H, G4), lambda k: (0, 0)),
        ],
        out_specs=(
            pl.BlockSpec((S, B, D_H), lambda k: (k, 0, 0)),
            pl.BlockSpec((B, D_H), lambda k: (0, 0)),
        ),
        scratch_shapes=[
            pltpu.VMEM((B, D_H), jnp.float32),
            pltpu.VMEM((B, D_H), jnp.float32),
        ],
        compiler_params=pltpu.CompilerParams(
            dimension_semantics=("arbitrary",),
        ),
        name="lstm_rec",
    )(pre, WhT)

    hx = outputs[-1]
    return outputs, (hx, cx)


# speedup vs baseline: 2.5280x; 1.0448x over previous
import jax
import jax.numpy as jnp
from jax.experimental import pallas as pl
from jax.experimental.pallas import tpu as pltpu

T, B, D_IN, D_H = 512, 64, 512, 512
G4 = 4 * D_H
BM = 1024
MB = T * B // BM
S = 16


def _proj_kernel(x_ref, wxt_ref, b_ref, o_ref):
    o_ref[...] = (
        jnp.dot(x_ref[...], wxt_ref[...], preferred_element_type=jnp.float32)
        + b_ref[...]
    ).astype(jnp.bfloat16)


def _rec_kernel(pre_ref, wht_ref, out_ref, cx_ref, h_ref, c_ref):
    k = pl.program_id(0)

    @pl.when(k == 0)
    def _():
        h_ref[...] = jnp.zeros_like(h_ref)
        c_ref[...] = jnp.zeros_like(c_ref)

    h = h_ref[...]
    c = c_ref[...]
    wht = wht_ref[...]
    for s in range(S):
        gates = pre_ref[s].astype(jnp.float32) + jnp.dot(
            h.astype(jnp.bfloat16), wht, preferred_element_type=jnp.float32
        )
        f = jax.nn.sigmoid(gates[:, 0 * D_H : 1 * D_H])
        i = jax.nn.sigmoid(gates[:, 1 * D_H : 2 * D_H])
        g = jnp.tanh(gates[:, 2 * D_H : 3 * D_H])
        o = jax.nn.sigmoid(gates[:, 3 * D_H : 4 * D_H])
        c = f * c + i * g
        h = o * jnp.tanh(c)
        out_ref[s] = h
    h_ref[...] = h
    c_ref[...] = c

    @pl.when(k == T // S - 1)
    def _():
        cx_ref[...] = c


def kernel(inputs, Wf, bf, Wi, bi, Wg, bg, Wo, bo):
    W = jnp.concatenate([Wf, Wi, Wg, Wo], axis=0)
    WxT = W[:, :D_IN].T
    WhT = W[:, D_IN:].T
    b = jnp.concatenate([bf, bi, bg, bo]).reshape(1, G4)

    WhT = WhT.astype(jnp.bfloat16)
    X = inputs.reshape(T * B, D_IN)
    pre = pl.pallas_call(
        _proj_kernel,
        out_shape=jax.ShapeDtypeStruct((T * B, G4), jnp.bfloat16),
        grid=(MB,),
        in_specs=[
            pl.BlockSpec((BM, D_IN), lambda m: (m, 0)),
            pl.BlockSpec((D_IN, G4), lambda m: (0, 0)),
            pl.BlockSpec((1, G4), lambda m: (0, 0)),
        ],
        out_specs=pl.BlockSpec((BM, G4), lambda m: (m, 0)),
        compiler_params=pltpu.CompilerParams(
            dimension_semantics=("parallel",),
        ),
        name="lstm_proj",
    )(X, WxT, b)
    pre = pre.reshape(T, B, G4)

    outputs, cx = pl.pallas_call(
        _rec_kernel,
        out_shape=(
            jax.ShapeDtypeStruct((T, B, D_H), jnp.float32),
            jax.ShapeDtypeStruct((B, D_H), jnp.float32),
        ),
        grid=(T // S,),
        in_specs=[
            pl.BlockSpec((S, B, G4), lambda k: (k, 0, 0)),
            pl.BlockSpec((D_H, G4), lambda k: (0, 0)),
        ],
        out_specs=(
            pl.BlockSpec((S, B, D_H), lambda k: (k, 0, 0)),
            pl.BlockSpec((B, D_H), lambda k: (0, 0)),
        ),
        scratch_shapes=[
            pltpu.VMEM((B, D_H), jnp.float32),
            pltpu.VMEM((B, D_H), jnp.float32),
        ],
        compiler_params=pltpu.CompilerParams(
            dimension_semantics=("arbitrary",),
        ),
        name="lstm_rec",
    )(pre, WhT)

    hx = outputs[-1]
    return outputs, (hx, cx)


# speedup vs baseline: 2.6581x; 1.0515x over previous
import jax
import jax.numpy as jnp
from jax.experimental import pallas as pl
from jax.experimental.pallas import tpu as pltpu

T, B, D_IN, D_H = 512, 64, 512, 512
G4 = 4 * D_H
BM = 1024
MB = T * B // BM
S = 16

_DN_T = (((1,), (1,)), ((), ()))


def _proj_kernel(x_ref, wf_ref, wi_ref, wg_ref, wo_ref, b_ref, o_ref):
    x = x_ref[...]
    for q, w_ref in enumerate((wf_ref, wi_ref, wg_ref, wo_ref)):
        o_ref[:, q * D_H : (q + 1) * D_H] = (
            jax.lax.dot_general(
                x, w_ref[...], _DN_T, preferred_element_type=jnp.float32
            )
            + b_ref[0, q * D_H : (q + 1) * D_H]
        ).astype(jnp.bfloat16)


def _rec_kernel(
    pre_ref, wf_ref, wi_ref, wg_ref, wo_ref,
    out_ref, cx_ref, hx_ref, h_ref, c_ref, wht_ref,
):
    k = pl.program_id(0)

    @pl.when(k == 0)
    def _():
        h_ref[...] = jnp.zeros_like(h_ref)
        c_ref[...] = jnp.zeros_like(c_ref)
        for q, w_ref in enumerate((wf_ref, wi_ref, wg_ref, wo_ref)):
            wht_ref[:, q * D_H : (q + 1) * D_H] = (
                w_ref[...].T.astype(jnp.bfloat16)
            )

    h = h_ref[...]
    c = c_ref[...]
    wht = wht_ref[...]
    for s in range(S):
        gates = pre_ref[s].astype(jnp.float32) + jnp.dot(
            h.astype(jnp.bfloat16), wht, preferred_element_type=jnp.float32
        )
        f = jax.nn.sigmoid(gates[:, 0 * D_H : 1 * D_H])
        i = jax.nn.sigmoid(gates[:, 1 * D_H : 2 * D_H])
        g = jnp.tanh(gates[:, 2 * D_H : 3 * D_H])
        o = jax.nn.sigmoid(gates[:, 3 * D_H : 4 * D_H])
        c = f * c + i * g
        h = o * jnp.tanh(c)
        out_ref[s] = h
    h_ref[...] = h
    c_ref[...] = c

    @pl.when(k == T // S - 1)
    def _():
        cx_ref[...] = c
        hx_ref[...] = h


def kernel(inputs, Wf, bf, Wi, bi, Wg, bg, Wo, bo):
    b = jnp.concatenate([bf, bi, bg, bo]).reshape(1, G4)
    X = inputs.reshape(T * B, D_IN)

    w_x_spec = pl.BlockSpec((D_H, D_IN), lambda m: (0, 0))
    pre = pl.pallas_call(
        _proj_kernel,
        out_shape=jax.ShapeDtypeStruct((T * B, G4), jnp.bfloat16),
        grid=(MB,),
        in_specs=[
            pl.BlockSpec((BM, D_IN), lambda m: (m, 0)),
            w_x_spec, w_x_spec, w_x_spec, w_x_spec,
            pl.BlockSpec((1, G4), lambda m: (0, 0)),
        ],
        out_specs=pl.BlockSpec((BM, G4), lambda m: (m, 0)),
        compiler_params=pltpu.CompilerParams(
            dimension_semantics=("parallel",),
        ),
        name="lstm_proj",
    )(X, Wf, Wi, Wg, Wo, b)
    pre = pre.reshape(T, B, G4)

    w_h_spec = pl.BlockSpec((D_H, D_H), lambda k: (0, 1))
    outputs, cx, hx = pl.pallas_call(
        _rec_kernel,
        out_shape=(
            jax.ShapeDtypeStruct((T, B, D_H), jnp.float32),
            jax.ShapeDtypeStruct((B, D_H), jnp.float32),
            jax.ShapeDtypeStruct((B, D_H), jnp.float32),
        ),
        grid=(T // S,),
        in_specs=[
            pl.BlockSpec((S, B, G4), lambda k: (k, 0, 0)),
            w_h_spec, w_h_spec, w_h_spec, w_h_spec,
        ],
        out_specs=(
            pl.BlockSpec((S, B, D_H), lambda k: (k, 0, 0)),
            pl.BlockSpec((B, D_H), lambda k: (0, 0)),
            pl.BlockSpec((B, D_H), lambda k: (0, 0)),
        ),
        scratch_shapes=[
            pltpu.VMEM((B, D_H), jnp.float32),
            pltpu.VMEM((B, D_H), jnp.float32),
            pltpu.VMEM((D_H, G4), jnp.bfloat16),
        ],
        compiler_params=pltpu.CompilerParams(
            dimension_semantics=("arbitrary",),
        ),
        name="lstm_rec",
    )(pre, Wf, Wi, Wg, Wo)

    return outputs, (hx, cx)


# speedup vs baseline: 2.7959x; 1.0518x over previous
import jax
import jax.numpy as jnp
from jax.experimental import pallas as pl
from jax.experimental.pallas import tpu as pltpu

T, B, D_IN, D_H = 512, 64, 512, 512
G4 = 4 * D_H
S = 16

_DN_T = (((1,), (1,)), ((), ()))


def _lstm_kernel(
    x_ref, wf_ref, wi_ref, wg_ref, wo_ref, b_ref,
    out_ref, cx_ref, hx_ref,
    h_ref, c_ref, wht_ref, pre_ref,
):
    k = pl.program_id(0)
    w_refs = (wf_ref, wi_ref, wg_ref, wo_ref)

    @pl.when(k == 0)
    def _():
        h_ref[...] = jnp.zeros_like(h_ref)
        c_ref[...] = jnp.zeros_like(c_ref)
        for q, w_ref in enumerate(w_refs):
            wht_ref[:, q * D_H : (q + 1) * D_H] = (
                w_ref[:, D_IN:][...].T.astype(jnp.bfloat16)
            )

    x2 = x_ref[...].reshape(S * B, D_IN)
    for q, w_ref in enumerate(w_refs):
        pre_ref[:, q * D_H : (q + 1) * D_H] = (
            jax.lax.dot_general(
                x2, w_ref[:, :D_IN][...], _DN_T,
                preferred_element_type=jnp.float32,
            )
            + b_ref[0, q * D_H : (q + 1) * D_H]
        )

    h = h_ref[...]
    c = c_ref[...]
    wht = wht_ref[...]
    for s in range(S):
        gates = pre_ref[s * B : (s + 1) * B, :] + jnp.dot(
            h.astype(jnp.bfloat16), wht, preferred_element_type=jnp.float32
        )
        f = jax.nn.sigmoid(gates[:, 0 * D_H : 1 * D_H])
        i = jax.nn.sigmoid(gates[:, 1 * D_H : 2 * D_H])
        g = jnp.tanh(gates[:, 2 * D_H : 3 * D_H])
        o = jax.nn.sigmoid(gates[:, 3 * D_H : 4 * D_H])
        c = f * c + i * g
        h = o * jnp.tanh(c)
        out_ref[s] = h
    h_ref[...] = h
    c_ref[...] = c

    @pl.when(k == T // S - 1)
    def _():
        cx_ref[...] = c
        hx_ref[...] = h


def kernel(inputs, Wf, bf, Wi, bi, Wg, bg, Wo, bo):
    b = jnp.concatenate([bf, bi, bg, bo]).reshape(1, G4)

    w_spec = pl.BlockSpec((D_H, D_IN + D_H), lambda k: (0, 0))
    outputs, cx, hx = pl.pallas_call(
        _lstm_kernel,
        out_shape=(
            jax.ShapeDtypeStruct((T, B, D_H), jnp.float32),
            jax.ShapeDtypeStruct((B, D_H), jnp.float32),
            jax.ShapeDtypeStruct((B, D_H), jnp.float32),
        ),
        grid=(T // S,),
        in_specs=[
            pl.BlockSpec((S, B, D_IN), lambda k: (k, 0, 0)),
            w_spec, w_spec, w_spec, w_spec,
            pl.BlockSpec((1, G4), lambda k: (0, 0)),
        ],
        out_specs=(
            pl.BlockSpec((S, B, D_H), lambda k: (k, 0, 0)),
            pl.BlockSpec((B, D_H), lambda k: (0, 0)),
            pl.BlockSpec((B, D_H), lambda k: (0, 0)),
        ),
        scratch_shapes=[
            pltpu.VMEM((B, D_H), jnp.float32),
            pltpu.VMEM((B, D_H), jnp.float32),
            pltpu.VMEM((D_H, G4), jnp.bfloat16),
            pltpu.VMEM((S * B, G4), jnp.float32),
        ],
        compiler_params=pltpu.CompilerParams(
            dimension_semantics=("arbitrary",),
        ),
        name="lstm_fused",
    )(inputs, Wf, Wi, Wg, Wo, b)

    return outputs, (hx, cx)
